# baseline (speedup 1.0000x reference)
import functools

import jax
import jax.numpy as jnp
from jax import lax
from jax.experimental import pallas as pl
from jax.experimental.pallas import tpu as pltpu
from jax.experimental.pallas import tpu_sc as plsc

_B = 16
_G = 76
_GG = _G * _G
_NA = 3
_NATTR = 85
_STRIDE = 8.0
_AW = (116.0, 156.0, 373.0)
_AH = (90.0, 198.0, 326.0)

_TR = 8
_HP = 304
_NT = 10
_NCHUNKS = _B * _NA * _NT
_NW = 32


def _sigmoid(v):
    return 1.0 / (1.0 + jnp.exp(-v))


def _sc_yolo(x_hbm, out_hbm, in_v, out_v, sem):
    cid = lax.axis_index("c")
    sid = lax.axis_index("s")
    wid = sid * 2 + cid
    i16 = lax.broadcasted_iota(jnp.int32, (16,), 0)

    def chunk_body(i, carry):
        chunk = i * _NW + wid
        b = (chunk * 2185) >> 16
        rem = chunk - b * 30
        a = (rem * 6554) >> 16
        t = rem - a * 10
        gy0 = t * _TR

        pltpu.async_copy(
            x_hbm.at[b, pl.ds(a * _NATTR, _NATTR), pl.ds(gy0, _TR)],
            in_v, sem).wait()

        aw = jnp.where(a == 0, _AW[0], jnp.where(a == 1, _AW[1], _AW[2]))
        ah = jnp.where(a == 0, _AH[0], jnp.where(a == 1, _AH[1], _AH[2]))

        def do_half(h):
            for v in range(19):
                p = v * 16 + i16
                gyo = (p * 863) >> 16
                gx = p - gyo * _G
                j = gyo + (h * 4)
                gxf = gx.astype(jnp.float32)
                gyf = (gy0 + j).astype(jnp.float32)

                def ld(c_vec):
                    return plsc.load_gather(in_v, [c_vec, j, gx])

                c0 = jnp.full((16,), 0, jnp.int32)
                plsc.store_scatter(out_v, [p, c0],
                                   (_sigmoid(ld(c0)) + gxf) * _STRIDE)
                c1 = jnp.full((16,), 1, jnp.int32)
                plsc.store_scatter(out_v, [p, c1],
                                   (_sigmoid(ld(c1)) + gyf) * _STRIDE)
                c2 = jnp.full((16,), 2, jnp.int32)
                plsc.store_scatter(
                    out_v, [p, c2],
                    jnp.minimum(jnp.exp(ld(c2)), 1000.0) * aw)
                c3 = jnp.full((16,), 3, jnp.int32)
                plsc.store_scatter(
                    out_v, [p, c3],
                    jnp.minimum(jnp.exp(ld(c3)), 1000.0) * ah)

            @plsc.parallel_loop(4, _NATTR, unroll=4)
            def ch_body(c):
                cvec = jnp.full((16,), c, jnp.int32)
                for v in range(19):
                    p = v * 16 + i16
                    gyo = (p * 863) >> 16
                    gx = p - gyo * _G
                    j = gyo + (h * 4)
                    plsc.store_scatter(
                        out_v, [p, cvec],
                        _sigmoid(plsc.load_gather(in_v, [cvec, j, gx])))

            row0 = a * _GG + gy0 * _G + h * _HP
            pltpu.sync_copy(out_v, out_hbm.at[b, pl.ds(row0, _HP)])

        do_half(0)

        @pl.when(t < _NT - 1)
        def _half1():
            do_half(1)

        return carry

    lax.fori_loop(0, _NCHUNKS // _NW, chunk_body, 0)


def kernel(x, img_size):
    del img_size
    mesh = plsc.VectorSubcoreMesh(core_axis_name="c", subcore_axis_name="s")
    f = functools.partial(
        pl.kernel,
        mesh=mesh,
        out_type=jax.ShapeDtypeStruct((_B, _NA * _GG, _NATTR), jnp.float32),
        scratch_types=[
            pltpu.VMEM((_NATTR, _TR, _G), jnp.float32),
            pltpu.VMEM((_HP, _NATTR), jnp.float32),
            pltpu.SemaphoreType.DMA,
        ],
        compiler_params=pltpu.CompilerParams(
            use_tc_tiling_on_sc=True, needs_layout_passes=False),
    )(_sc_yolo)
    return f(x)


# speedup vs baseline: 1.5809x; 1.5809x over previous
import functools

import jax
import jax.numpy as jnp
import numpy as np
from jax import lax
from jax.experimental import pallas as pl
from jax.experimental.pallas import tpu as pltpu
from jax.experimental.pallas import tpu_sc as plsc

_B = 16
_SPLIT = 11
_G = 76
_GG = _G * _G
_NA = 3
_NATTR = 85
_STRIDE = 8.0
_AW = (116.0, 156.0, 373.0)
_AH = (90.0, 198.0, 326.0)
_ANCHORS_WH = np.array([[116.0, 90.0], [156.0, 198.0], [373.0, 326.0]],
                       dtype=np.float32)

_TR = 8
_HP = 304
_NT = 10
_NW = 32


def _tc_pane_kernel(x_ref, out_ref):
    v = x_ref[0]
    gg = v.shape[1] * v.shape[2]
    a = pl.program_id(1)
    aw = jnp.where(a == 0, _AW[0], jnp.where(a == 1, _AW[1], _AW[2]))
    ah = jnp.where(a == 0, _AH[0], jnp.where(a == 1, _AH[1], _AH[2]))
    sig = jax.nn.sigmoid(v)

    h = v[0:8]
    shp = h.shape
    row = lax.broadcasted_iota(jnp.int32, shp, 0)
    gy = lax.broadcasted_iota(jnp.int32, shp, 1).astype(jnp.float32)
    gx = lax.broadcasted_iota(jnp.int32, shp, 2).astype(jnp.float32)
    sig_h = sig[0:8]
    ex = jnp.minimum(jnp.exp(h), 1000.0) * jnp.where(row == 2, aw, ah)
    box = jnp.where(row < 2, (sig_h + jnp.where(row == 0, gx, gy)) * _STRIDE,
                    ex)
    head = jnp.where(row < 4, box, sig_h)
    res = jnp.concatenate([head, sig[8:]], axis=0).reshape(_NATTR, gg)

    eye = (lax.broadcasted_iota(jnp.int32, (_NATTR, _NATTR), 0)
           == lax.broadcasted_iota(jnp.int32, (_NATTR, _NATTR), 1)
           ).astype(jnp.float32)
    out_ref[0] = lax.dot_general(
        res, eye, (((0,), (0,)), ((), ())),
        preferred_element_type=jnp.float32)


def _tc_part(x):
    return pl.pallas_call(
        _tc_pane_kernel,
        grid=(_SPLIT, _NA),
        in_specs=[
            pl.BlockSpec((1, _NATTR, _G, _G), lambda b, a: (b, a, 0, 0)),
        ],
        out_specs=pl.BlockSpec((1, _GG, _NATTR), lambda b, a: (b, a, 0)),
        out_shape=jax.ShapeDtypeStruct((_SPLIT, _NA * _GG, _NATTR),
                                       jnp.float32),
    )(x)


_SCB = _B - _SPLIT
_NCHUNKS = _SCB * _NA * _NT


def _sigmoid(v):
    return 1.0 / (1.0 + jnp.exp(-v))


def _sc_yolo(x_hbm, out_hbm, in_v, out_v, sem):
    cid = lax.axis_index("c")
    sid = lax.axis_index("s")
    wid = sid * 2 + cid
    i16 = lax.broadcasted_iota(jnp.int32, (16,), 0)

    def chunk_body(i, carry):
        chunk = i * _NW + wid

        @pl.when(chunk < _NCHUNKS)
        def _do_chunk():
            b = (chunk * 2185) >> 16
            rem = chunk - b * 30
            a = (rem * 6554) >> 16
            t = rem - a * 10
            gy0 = t * _TR

            pltpu.async_copy(
                x_hbm.at[_SPLIT + b, pl.ds(a * _NATTR, _NATTR),
                         pl.ds(gy0, _TR)],
                in_v, sem).wait()

            aw = jnp.where(a == 0, _AW[0], jnp.where(a == 1, _AW[1], _AW[2]))
            ah = jnp.where(a == 0, _AH[0], jnp.where(a == 1, _AH[1], _AH[2]))

            def do_half(h):
                for v in range(19):
                    p = v * 16 + i16
                    gyo = (p * 863) >> 16
                    gx = p - gyo * _G
                    j = gyo + (h * 4)
                    gxf = gx.astype(jnp.float32)
                    gyf = (gy0 + j).astype(jnp.float32)

                    def ld(c_vec):
                        return plsc.load_gather(in_v, [c_vec, j, gx])

                    c0 = jnp.full((16,), 0, jnp.int32)
                    plsc.store_scatter(out_v, [p, c0],
                                       (_sigmoid(ld(c0)) + gxf) * _STRIDE)
                    c1 = jnp.full((16,), 1, jnp.int32)
                    plsc.store_scatter(out_v, [p, c1],
                                       (_sigmoid(ld(c1)) + gyf) * _STRIDE)
                    c2 = jnp.full((16,), 2, jnp.int32)
                    plsc.store_scatter(
                        out_v, [p, c2],
                        jnp.minimum(jnp.exp(ld(c2)), 1000.0) * aw)
                    c3 = jnp.full((16,), 3, jnp.int32)
                    plsc.store_scatter(
                        out_v, [p, c3],
                        jnp.minimum(jnp.exp(ld(c3)), 1000.0) * ah)

                @plsc.parallel_loop(4, _NATTR, unroll=2)
                def ch_body(c):
                    cvec = jnp.full((16,), c, jnp.int32)
                    for v in range(19):
                        p = v * 16 + i16
                        gyo = (p * 863) >> 16
                        gx = p - gyo * _G
                        j = gyo + (h * 4)
                        plsc.store_scatter(
                            out_v, [p, cvec],
                            _sigmoid(plsc.load_gather(in_v, [cvec, j, gx])))

                row0 = a * _GG + gy0 * _G + h * _HP
                pltpu.sync_copy(out_v, out_hbm.at[b, pl.ds(row0, _HP)])

            do_half(0)

            @pl.when(t < _NT - 1)
            def _half1():
                do_half(1)

        return carry

    lax.fori_loop(0, (_NCHUNKS + _NW - 1) // _NW, chunk_body, 0)


def _sc_part(x):
    mesh = plsc.VectorSubcoreMesh(core_axis_name="c", subcore_axis_name="s")
    f = functools.partial(
        pl.kernel,
        mesh=mesh,
        out_type=jax.ShapeDtypeStruct((_SCB, _NA * _GG, _NATTR), jnp.float32),
        scratch_types=[
            pltpu.VMEM((_NATTR, _TR, _G), jnp.float32),
            pltpu.VMEM((_HP, _NATTR), jnp.float32),
            pltpu.SemaphoreType.DMA,
        ],
        compiler_params=pltpu.CompilerParams(
            use_tc_tiling_on_sc=True, needs_layout_passes=False),
    )(_sc_yolo)
    return f(x)


def kernel(x, img_size):
    del img_size
    out_sc = _sc_part(x)
    out_tc = _tc_part(x)
    return jnp.concatenate([out_tc, out_sc], axis=0)


# speedup vs baseline: 2.1982x; 1.3905x over previous
import jax
import jax.numpy as jnp
import numpy as np
from jax import lax
from jax.experimental import pallas as pl
from jax.experimental.pallas import tpu as pltpu

_N_ANCHORS = 3
_N_CLS = 80
_N_ATTR = _N_CLS + 5
_ANCHORS_WH = np.array([[116.0, 90.0], [156.0, 198.0], [373.0, 326.0]],
                       dtype=np.float32)


def _yolo_pane_kernel(params_ref, x_ref, out_ref):
    stride = params_ref[0]
    a = pl.program_id(1)
    aw = jnp.where(a == 0, params_ref[1],
                   jnp.where(a == 1, params_ref[3], params_ref[5]))
    ah = jnp.where(a == 0, params_ref[2],
                   jnp.where(a == 1, params_ref[4], params_ref[6]))

    v = x_ref[0]
    gg = v.shape[1] * v.shape[2]
    sig = jax.nn.sigmoid(v)

    h = v[0:8]
    shp = h.shape
    row = lax.broadcasted_iota(jnp.int32, shp, 0)
    gy = lax.broadcasted_iota(jnp.int32, shp, 1).astype(jnp.float32)
    gx = lax.broadcasted_iota(jnp.int32, shp, 2).astype(jnp.float32)
    sig_h = sig[0:8]
    ex = jnp.minimum(jnp.exp(h), 1000.0) * jnp.where(row == 2, aw, ah)
    box = jnp.where(row < 2, (sig_h + jnp.where(row == 0, gx, gy)) * stride,
                    ex)
    head = jnp.where(row < 4, box, sig_h)
    res = jnp.concatenate([head, sig[8:]], axis=0).reshape(_N_ATTR, gg)

    eye = (lax.broadcasted_iota(jnp.int32, (_N_ATTR, _N_ATTR), 0)
           == lax.broadcasted_iota(jnp.int32, (_N_ATTR, _N_ATTR), 1)
           ).astype(jnp.float32)
    out_ref[0] = lax.dot_general(
        res, eye, (((0,), (0,)), ((), ())),
        preferred_element_type=jnp.float32)


def kernel(x, img_size):
    B = x.shape[0]
    G = x.shape[2]
    GG = G * G
    nA = _N_ANCHORS

    stride = jnp.float32(img_size) / jnp.float32(G)
    anch = jnp.asarray(_ANCHORS_WH)
    anch_eff = (anch / stride) * stride
    params = jnp.concatenate([
        stride[None], anch_eff.reshape(-1)
    ]).astype(jnp.float32)

    return pl.pallas_call(
        _yolo_pane_kernel,
        grid=(B, nA),
        in_specs=[
            pl.BlockSpec(memory_space=pltpu.SMEM),
            pl.BlockSpec((1, _N_ATTR, G, G), lambda b, a: (b, a, 0, 0)),
        ],
        out_specs=pl.BlockSpec((1, GG, _N_ATTR), lambda b, a: (b, a, 0)),
        out_shape=jax.ShapeDtypeStruct((B, nA * GG, _N_ATTR), jnp.float32),
        compiler_params=pltpu.CompilerParams(
            allow_input_fusion=[False, True]),
    )(params, x)


# speedup vs baseline: 2.2001x; 1.0008x over previous
import jax
import jax.numpy as jnp
import numpy as np
from jax import lax
from jax.experimental import pallas as pl
from jax.experimental.pallas import tpu as pltpu

_N_ANCHORS = 3
_N_CLS = 80
_N_ATTR = _N_CLS + 5
_ANCHORS_WH = np.array([[116.0, 90.0], [156.0, 198.0], [373.0, 326.0]],
                       dtype=np.float32)


def _yolo_pane_kernel(params_ref, x_ref, out_ref):
    stride = params_ref[0]
    a = pl.program_id(1)
    aw = jnp.where(a == 0, params_ref[1],
                   jnp.where(a == 1, params_ref[3], params_ref[5]))
    ah = jnp.where(a == 0, params_ref[2],
                   jnp.where(a == 1, params_ref[4], params_ref[6]))

    v = x_ref[0]
    gg = v.shape[1] * v.shape[2]
    sig = jax.nn.sigmoid(v)

    h = v[0:8]
    shp = h.shape
    row = lax.broadcasted_iota(jnp.int32, shp, 0)
    gy = lax.broadcasted_iota(jnp.int32, shp, 1).astype(jnp.float32)
    gx = lax.broadcasted_iota(jnp.int32, shp, 2).astype(jnp.float32)
    sig_h = sig[0:8]
    ex = jnp.minimum(jnp.exp(h), 1000.0) * jnp.where(row == 2, aw, ah)
    box = jnp.where(row < 2, (sig_h + jnp.where(row == 0, gx, gy)) * stride,
                    ex)
    head = jnp.where(row < 4, box, sig_h)
    res = jnp.concatenate([head, sig[8:]], axis=0).reshape(_N_ATTR, gg)

    eye = (lax.broadcasted_iota(jnp.int32, (_N_ATTR, _N_ATTR), 0)
           == lax.broadcasted_iota(jnp.int32, (_N_ATTR, _N_ATTR), 1)
           ).astype(jnp.float32)
    out_ref[0] = lax.dot_general(
        res, eye, (((0,), (0,)), ((), ())),
        preferred_element_type=jnp.float32)


def kernel(x, img_size):
    B = x.shape[0]
    G = x.shape[2]
    GG = G * G
    nA = _N_ANCHORS

    stride = jnp.float32(img_size) / jnp.float32(G)
    anch = jnp.asarray(_ANCHORS_WH)
    anch_eff = (anch / stride) * stride
    params = jnp.concatenate([
        stride[None], anch_eff.reshape(-1)
    ]).astype(jnp.float32)

    return pl.pallas_call(
        _yolo_pane_kernel,
        grid=(B, nA),
        in_specs=[
            pl.BlockSpec(memory_space=pltpu.SMEM),
            pl.BlockSpec((1, _N_ATTR, G, G), lambda b, a: (b, a, 0, 0)),
        ],
        out_specs=pl.BlockSpec((1, GG, _N_ATTR), lambda b, a: (b, a, 0)),
        out_shape=jax.ShapeDtypeStruct((B, nA * GG, _N_ATTR), jnp.float32),
    )(params, x)
